# initial kernel scaffold (unmeasured)
import jax
import jax.numpy as jnp
from jax import lax
from jax.experimental import pallas as pl
from jax.experimental.pallas import tpu as pltpu

N_DEV = 16


def kernel(x, w_mat, scale_x, scale_w):
    m_per, k = x.shape
    _, n_per = w_mat.shape
    m_total = N_DEV * m_per

    def body(x_ref, w_ref, sx_ref, sw_ref, out_ref, comm_ref, send_sems, recv_sems):
        my = lax.axis_index("i")
        left = lax.rem(my + N_DEV - 1, N_DEV)
        right = lax.rem(my + 1, N_DEV)

        barrier_sem = pltpu.get_barrier_semaphore()
        for nbr in (left, right):
            pl.semaphore_signal(
                barrier_sem, inc=1,
                device_id=(nbr,), device_id_type=pl.DeviceIdType.MESH,
            )
        pl.semaphore_wait(barrier_sem, 2)

        scale = sx_ref[0] * sw_ref[0]

        def gemm_store(chunk, origin):
            acc = lax.dot_general(
                chunk, w_ref[...],
                (((1,), (0,)), ((), ())),
                preferred_element_type=jnp.float32,
            )
            y = acc * scale
            z = y / (1.0 + jnp.exp(-jnp.clip(y, -60.0, 60.0)))
            out_ref[pl.ds(origin * m_per, m_per), :] = z

        gemm_store(x_ref[...], my)

        for h in range(N_DEV - 1):
            src = x_ref if h == 0 else comm_ref.at[h - 1]
            rdma = pltpu.make_async_remote_copy(
                src_ref=src,
                dst_ref=comm_ref.at[h],
                send_sem=send_sems.at[h],
                recv_sem=recv_sems.at[h],
                device_id=(right,),
                device_id_type=pl.DeviceIdType.MESH,
            )
            rdma.start()
            rdma.wait()
            origin = lax.rem(my + N_DEV - (h + 1), N_DEV)
            gemm_store(comm_ref[h], origin)

    return pl.pallas_call(
        body,
        out_shape=jax.ShapeDtypeStruct((m_total, n_per), jnp.float32),
        in_specs=[
            pl.BlockSpec(memory_space=pltpu.VMEM),
            pl.BlockSpec(memory_space=pltpu.VMEM),
            pl.BlockSpec(memory_space=pltpu.SMEM),
            pl.BlockSpec(memory_space=pltpu.SMEM),
        ],
        out_specs=pl.BlockSpec(memory_space=pltpu.VMEM),
        scratch_shapes=[
            pltpu.VMEM((N_DEV - 1, m_per, k), x.dtype),
            pltpu.SemaphoreType.DMA((N_DEV - 1,)),
            pltpu.SemaphoreType.DMA((N_DEV - 1,)),
        ],
        compiler_params=pltpu.CompilerParams(collective_id=0),
    )(x, w_mat, scale_x, scale_w)


# baseline (device time: 224872 ns/iter reference)
import jax
import jax.numpy as jnp
from jax import lax
from jax.experimental import pallas as pl
from jax.experimental.pallas import tpu as pltpu

N_DEV = 16


def kernel(x, w_mat, scale_x, scale_w):
    m_per, k = x.shape
    _, n_per = w_mat.shape
    m_total = N_DEV * m_per

    x = x.astype(jnp.float8_e4m3fn)
    w_mat = w_mat.astype(jnp.float8_e4m3fn)

    def body(x_ref, w_ref, sx_ref, sw_ref, out_ref, comm_ref, send_sems, recv_sems):
        my = lax.axis_index("i")
        left = lax.rem(my + N_DEV - 1, N_DEV)
        right = lax.rem(my + 1, N_DEV)

        barrier_sem = pltpu.get_barrier_semaphore()
        for nbr in (left, right):
            pl.semaphore_signal(
                barrier_sem, inc=1,
                device_id=(nbr,), device_id_type=pl.DeviceIdType.MESH,
            )
        pl.semaphore_wait(barrier_sem, 2)

        scale = sx_ref[0] * sw_ref[0]

        def gemm_store(chunk, origin):
            acc = lax.dot_general(
                chunk, w_ref[...],
                (((1,), (0,)), ((), ())),
                preferred_element_type=jnp.float32,
            )
            y = acc * scale
            z = y / (1.0 + jnp.exp(-jnp.clip(y, -60.0, 60.0)))
            out_ref[pl.ds(origin * m_per, m_per), :] = z

        gemm_store(x_ref[...], my)

        for h in range(N_DEV - 1):
            src = x_ref if h == 0 else comm_ref.at[h - 1]
            rdma = pltpu.make_async_remote_copy(
                src_ref=src,
                dst_ref=comm_ref.at[h],
                send_sem=send_sems.at[h],
                recv_sem=recv_sems.at[h],
                device_id=(right,),
                device_id_type=pl.DeviceIdType.MESH,
            )
            rdma.start()
            rdma.wait()
            origin = lax.rem(my + N_DEV - (h + 1), N_DEV)
            gemm_store(comm_ref[h], origin)

    return pl.pallas_call(
        body,
        out_shape=jax.ShapeDtypeStruct((m_total, n_per), jnp.float32),
        in_specs=[
            pl.BlockSpec(memory_space=pltpu.VMEM),
            pl.BlockSpec(memory_space=pltpu.VMEM),
            pl.BlockSpec(memory_space=pltpu.SMEM),
            pl.BlockSpec(memory_space=pltpu.SMEM),
        ],
        out_specs=pl.BlockSpec(memory_space=pltpu.VMEM),
        scratch_shapes=[
            pltpu.VMEM((N_DEV - 1, m_per, k), x.dtype),
            pltpu.SemaphoreType.DMA((N_DEV - 1,)),
            pltpu.SemaphoreType.DMA((N_DEV - 1,)),
        ],
        compiler_params=pltpu.CompilerParams(collective_id=0),
    )(x, w_mat, scale_x, scale_w)


# device time: 124331 ns/iter; 1.8087x vs baseline; 1.8087x over previous
import jax
import jax.numpy as jnp
from jax import lax
from jax.experimental import pallas as pl
from jax.experimental.pallas import tpu as pltpu

N_DEV = 16
N_CW = 8
N_CCW = 7

PERM = (0, 4, 8, 12, 13, 9, 5, 1, 2, 6, 10, 14, 15, 11, 7, 3)
INV = tuple(PERM.index(p) for p in range(N_DEV))


def kernel(x, w_mat, scale_x, scale_w):
    m_per, k = x.shape
    _, n_per = w_mat.shape
    m_total = N_DEV * m_per

    x = x.astype(jnp.float8_e4m3fn)
    w_mat = w_mat.astype(jnp.float8_e4m3fn)

    my = lax.axis_index("i")
    perm = jnp.asarray(PERM, jnp.int32)
    inv = jnp.asarray(INV, jnp.int32)
    r = inv[my]
    nbrs = jnp.stack([perm[(r - 1) % N_DEV], perm[(r + 1) % N_DEV]])
    cw_org = perm[(r - 1 - jnp.arange(N_CW, dtype=jnp.int32)) % N_DEV]
    ccw_org = perm[(r + 1 + jnp.arange(N_CCW, dtype=jnp.int32)) % N_DEV]

    def body(x_ref, w_ref, sx_ref, sw_ref, nbr_ref, cw_org_ref, ccw_org_ref,
             out_ref, cw_ref, ccw_ref, cw_send, cw_recv, ccw_send, ccw_recv):
        left = nbr_ref[0]
        right = nbr_ref[1]

        barrier_sem = pltpu.get_barrier_semaphore()
        for nbr in (left, right):
            pl.semaphore_signal(
                barrier_sem, inc=1,
                device_id=(nbr,), device_id_type=pl.DeviceIdType.MESH,
            )
        pl.semaphore_wait(barrier_sem, 2)

        scale = sx_ref[0] * sw_ref[0]

        def gemm_store(chunk, origin):
            acc = lax.dot_general(
                chunk, w_ref[...],
                (((1,), (0,)), ((), ())),
                preferred_element_type=jnp.float32,
            )
            y = acc * scale
            z = y / (1.0 + jnp.exp(-jnp.clip(y, -60.0, 60.0)))
            out_ref[pl.ds(origin * m_per, m_per), :] = z

        def mk(src, dst_buf, slot, send_sems, recv_sems, dev):
            return pltpu.make_async_remote_copy(
                src_ref=src,
                dst_ref=dst_buf.at[slot],
                send_sem=send_sems.at[slot],
                recv_sem=recv_sems.at[slot],
                device_id=(dev,),
                device_id_type=pl.DeviceIdType.MESH,
            )

        cw_rdmas = [mk(x_ref, cw_ref, 0, cw_send, cw_recv, right)]
        ccw_rdmas = [mk(x_ref, ccw_ref, 0, ccw_send, ccw_recv, left)]
        cw_rdmas[0].start()
        ccw_rdmas[0].start()
        gemm_store(x_ref[...], lax.axis_index("i"))

        for h in range(N_CW):
            cw_rdmas[h].wait_recv()
            if h + 1 < N_CW:
                nxt = mk(cw_ref.at[h], cw_ref, h + 1, cw_send, cw_recv, right)
                nxt.start()
                cw_rdmas.append(nxt)
            if h < N_CCW:
                ccw_rdmas[h].wait_recv()
                if h + 1 < N_CCW:
                    nxt = mk(ccw_ref.at[h], ccw_ref, h + 1,
                             ccw_send, ccw_recv, left)
                    nxt.start()
                    ccw_rdmas.append(nxt)
            gemm_store(cw_ref[h], cw_org_ref[h])
            if h < N_CCW:
                gemm_store(ccw_ref[h], ccw_org_ref[h])

        for rd in cw_rdmas + ccw_rdmas:
            rd.wait_send()

    return pl.pallas_call(
        body,
        out_shape=jax.ShapeDtypeStruct((m_total, n_per), jnp.float32),
        in_specs=[
            pl.BlockSpec(memory_space=pltpu.VMEM),
            pl.BlockSpec(memory_space=pltpu.VMEM),
            pl.BlockSpec(memory_space=pltpu.SMEM),
            pl.BlockSpec(memory_space=pltpu.SMEM),
            pl.BlockSpec(memory_space=pltpu.SMEM),
            pl.BlockSpec(memory_space=pltpu.SMEM),
            pl.BlockSpec(memory_space=pltpu.SMEM),
        ],
        out_specs=pl.BlockSpec(memory_space=pltpu.VMEM),
        scratch_shapes=[
            pltpu.VMEM((N_CW, m_per, k), x.dtype),
            pltpu.VMEM((N_CCW, m_per, k), x.dtype),
            pltpu.SemaphoreType.DMA((N_CW,)),
            pltpu.SemaphoreType.DMA((N_CW,)),
            pltpu.SemaphoreType.DMA((N_CCW,)),
            pltpu.SemaphoreType.DMA((N_CCW,)),
        ],
        compiler_params=pltpu.CompilerParams(collective_id=0),
    )(x, w_mat, scale_x, scale_w, nbrs, cw_org, ccw_org)


# device time: 109031 ns/iter; 2.0625x vs baseline; 1.1403x over previous
import jax
import jax.numpy as jnp
from jax import lax
from jax.experimental import pallas as pl
from jax.experimental.pallas import tpu as pltpu

N_DEV = 16
N_HOPS = 8
SUBS = 2

PERM = (0, 4, 8, 12, 13, 9, 5, 1, 2, 6, 10, 14, 15, 11, 7, 3)
INV = tuple(PERM.index(p) for p in range(N_DEV))


def kernel(x, w_mat, scale_x, scale_w):
    m_per, k = x.shape
    _, n_per = w_mat.shape
    m_total = N_DEV * m_per
    rows = m_per // SUBS

    x = x.astype(jnp.float8_e4m3fn)
    w_mat = w_mat.astype(jnp.float8_e4m3fn)

    my = lax.axis_index("i")
    perm = jnp.asarray(PERM, jnp.int32)
    inv = jnp.asarray(INV, jnp.int32)
    r = inv[my]
    nbrs = jnp.stack([perm[(r - 1) % N_DEV], perm[(r + 1) % N_DEV]])
    cw_org = perm[(r - 1 - jnp.arange(N_HOPS, dtype=jnp.int32)) % N_DEV]
    ccw_org = perm[(r + 1 + jnp.arange(N_HOPS, dtype=jnp.int32)) % N_DEV]

    def cw_has(h, s):
        return h < N_HOPS - 1 or s < SUBS // 2

    def ccw_has(h, s):
        return h < N_HOPS - 1 or s >= SUBS // 2

    def body(x_ref, w_ref, sx_ref, sw_ref, nbr_ref, cw_org_ref, ccw_org_ref,
             out_ref, cw_ref, ccw_ref, cw_send, cw_recv, ccw_send, ccw_recv):
        left = nbr_ref[0]
        right = nbr_ref[1]

        barrier_sem = pltpu.get_barrier_semaphore()
        for nbr in (left, right):
            pl.semaphore_signal(
                barrier_sem, inc=1,
                device_id=(nbr,), device_id_type=pl.DeviceIdType.MESH,
            )
        pl.semaphore_wait(barrier_sem, 2)

        scale = sx_ref[0] * sw_ref[0]

        def gemm_store(chunk, row_start):
            acc = lax.dot_general(
                chunk, w_ref[...],
                (((1,), (0,)), ((), ())),
                preferred_element_type=jnp.float32,
            )
            y = acc * scale
            z = y / (1.0 + jnp.exp(-jnp.clip(y, -60.0, 60.0)))
            out_ref[pl.ds(row_start, chunk.shape[0]), :] = z

        def mk(src, dst_buf, h, s, send_sems, recv_sems, dev):
            return pltpu.make_async_remote_copy(
                src_ref=src,
                dst_ref=dst_buf.at[h, s],
                send_sem=send_sems.at[h, s],
                recv_sem=recv_sems.at[h, s],
                device_id=(dev,),
                device_id_type=pl.DeviceIdType.MESH,
            )

        sends = []

        for s in range(SUBS):
            piece = x_ref.at[pl.ds(s * rows, rows)]
            rd = mk(piece, cw_ref, 0, s, cw_send, cw_recv, right)
            rd.start()
            sends.append(rd)
            rd = mk(piece, ccw_ref, 0, s, ccw_send, ccw_recv, left)
            rd.start()
            sends.append(rd)
        gemm_store(x_ref[...], lax.axis_index("i") * m_per)

        def recv_cw(h, s):
            return mk(x_ref.at[pl.ds(0, rows)], cw_ref, h, s,
                      cw_send, cw_recv, right)

        def recv_ccw(h, s):
            return mk(x_ref.at[pl.ds(0, rows)], ccw_ref, h, s,
                      ccw_send, ccw_recv, left)

        for h in range(N_HOPS):
            for s in range(SUBS):
                if cw_has(h, s):
                    recv_cw(h, s).wait_recv()
                    if h + 1 < N_HOPS and cw_has(h + 1, s):
                        rd = mk(cw_ref.at[h, s], cw_ref, h + 1, s,
                                cw_send, cw_recv, right)
                        rd.start()
                        sends.append(rd)
                if ccw_has(h, s):
                    recv_ccw(h, s).wait_recv()
                    if h + 1 < N_HOPS and ccw_has(h + 1, s):
                        rd = mk(ccw_ref.at[h, s], ccw_ref, h + 1, s,
                                ccw_send, ccw_recv, left)
                        rd.start()
                        sends.append(rd)
            for s in range(SUBS):
                if cw_has(h, s):
                    gemm_store(cw_ref[h, s], cw_org_ref[h] * m_per + s * rows)
                if ccw_has(h, s):
                    gemm_store(ccw_ref[h, s], ccw_org_ref[h] * m_per + s * rows)

        for rd in sends:
            rd.wait_send()

    return pl.pallas_call(
        body,
        out_shape=jax.ShapeDtypeStruct((m_total, n_per), jnp.float32),
        in_specs=[
            pl.BlockSpec(memory_space=pltpu.VMEM),
            pl.BlockSpec(memory_space=pltpu.VMEM),
            pl.BlockSpec(memory_space=pltpu.SMEM),
            pl.BlockSpec(memory_space=pltpu.SMEM),
            pl.BlockSpec(memory_space=pltpu.SMEM),
            pl.BlockSpec(memory_space=pltpu.SMEM),
            pl.BlockSpec(memory_space=pltpu.SMEM),
        ],
        out_specs=pl.BlockSpec(memory_space=pltpu.VMEM),
        scratch_shapes=[
            pltpu.VMEM((N_HOPS, SUBS, rows, k), x.dtype),
            pltpu.VMEM((N_HOPS, SUBS, rows, k), x.dtype),
            pltpu.SemaphoreType.DMA((N_HOPS, SUBS)),
            pltpu.SemaphoreType.DMA((N_HOPS, SUBS)),
            pltpu.SemaphoreType.DMA((N_HOPS, SUBS)),
            pltpu.SemaphoreType.DMA((N_HOPS, SUBS)),
        ],
        compiler_params=pltpu.CompilerParams(collective_id=0),
    )(x, w_mat, scale_x, scale_w, nbrs, cw_org, ccw_org)


# device time: 107804 ns/iter; 2.0859x vs baseline; 1.0114x over previous
import jax
import jax.numpy as jnp
from jax import lax
from jax.experimental import pallas as pl
from jax.experimental.pallas import tpu as pltpu

N_DEV = 16
N_HOPS = 8
SUBS = 4

PERM = (0, 4, 8, 12, 13, 9, 5, 1, 2, 6, 10, 14, 15, 11, 7, 3)
INV = tuple(PERM.index(p) for p in range(N_DEV))


def kernel(x, w_mat, scale_x, scale_w):
    m_per, k = x.shape
    _, n_per = w_mat.shape
    m_total = N_DEV * m_per
    rows = m_per // SUBS

    x = x.astype(jnp.float8_e4m3fn)
    w_mat = w_mat.astype(jnp.float8_e4m3fn)

    my = lax.axis_index("i")
    perm = jnp.asarray(PERM, jnp.int32)
    inv = jnp.asarray(INV, jnp.int32)
    r = inv[my]
    nbrs = jnp.stack([perm[(r - 1) % N_DEV], perm[(r + 1) % N_DEV]])
    cw_org = perm[(r - 1 - jnp.arange(N_HOPS, dtype=jnp.int32)) % N_DEV]
    ccw_org = perm[(r + 1 + jnp.arange(N_HOPS, dtype=jnp.int32)) % N_DEV]

    def cw_has(h, s):
        return h < N_HOPS - 1 or s < SUBS // 2

    def ccw_has(h, s):
        return h < N_HOPS - 1 or s >= SUBS // 2

    def body(x_ref, w_ref, sx_ref, sw_ref, nbr_ref, cw_org_ref, ccw_org_ref,
             out_ref, cw_ref, ccw_ref, cw_send, cw_recv, ccw_send, ccw_recv):
        left = nbr_ref[0]
        right = nbr_ref[1]

        barrier_sem = pltpu.get_barrier_semaphore()
        for nbr in (left, right):
            pl.semaphore_signal(
                barrier_sem, inc=1,
                device_id=(nbr,), device_id_type=pl.DeviceIdType.MESH,
            )
        pl.semaphore_wait(barrier_sem, 2)

        scale = sx_ref[0] * sw_ref[0]

        def gemm_store(chunk, row_start):
            acc = lax.dot_general(
                chunk, w_ref[...],
                (((1,), (0,)), ((), ())),
                preferred_element_type=jnp.float32,
            )
            y = acc * scale
            z = y / (1.0 + jnp.exp(-jnp.clip(y, -60.0, 60.0)))
            out_ref[pl.ds(row_start, chunk.shape[0]), :] = z

        def mk(src, dst_buf, h, s, send_sems, recv_sems, dev):
            return pltpu.make_async_remote_copy(
                src_ref=src,
                dst_ref=dst_buf.at[h, s],
                send_sem=send_sems.at[h, s],
                recv_sem=recv_sems.at[h, s],
                device_id=(dev,),
                device_id_type=pl.DeviceIdType.MESH,
            )

        sends = []

        for s in range(SUBS):
            piece = x_ref.at[pl.ds(s * rows, rows)]
            rd = mk(piece, cw_ref, 0, s, cw_send, cw_recv, right)
            rd.start()
            sends.append(rd)
            rd = mk(piece, ccw_ref, 0, s, ccw_send, ccw_recv, left)
            rd.start()
            sends.append(rd)
        gemm_store(x_ref[...], lax.axis_index("i") * m_per)

        def recv_cw(h, s):
            return mk(x_ref.at[pl.ds(0, rows)], cw_ref, h, s,
                      cw_send, cw_recv, right)

        def recv_ccw(h, s):
            return mk(x_ref.at[pl.ds(0, rows)], ccw_ref, h, s,
                      ccw_send, ccw_recv, left)

        for h in range(N_HOPS):
            for s in range(SUBS):
                if cw_has(h, s):
                    recv_cw(h, s).wait_recv()
                    if h + 1 < N_HOPS and cw_has(h + 1, s):
                        rd = mk(cw_ref.at[h, s], cw_ref, h + 1, s,
                                cw_send, cw_recv, right)
                        rd.start()
                        sends.append(rd)
                if ccw_has(h, s):
                    recv_ccw(h, s).wait_recv()
                    if h + 1 < N_HOPS and ccw_has(h + 1, s):
                        rd = mk(ccw_ref.at[h, s], ccw_ref, h + 1, s,
                                ccw_send, ccw_recv, left)
                        rd.start()
                        sends.append(rd)
            for s in range(SUBS):
                if cw_has(h, s):
                    gemm_store(cw_ref[h, s], cw_org_ref[h] * m_per + s * rows)
                if ccw_has(h, s):
                    gemm_store(ccw_ref[h, s], ccw_org_ref[h] * m_per + s * rows)

        for rd in sends:
            rd.wait_send()

    return pl.pallas_call(
        body,
        out_shape=jax.ShapeDtypeStruct((m_total, n_per), jnp.float32),
        in_specs=[
            pl.BlockSpec(memory_space=pltpu.VMEM),
            pl.BlockSpec(memory_space=pltpu.VMEM),
            pl.BlockSpec(memory_space=pltpu.SMEM),
            pl.BlockSpec(memory_space=pltpu.SMEM),
            pl.BlockSpec(memory_space=pltpu.SMEM),
            pl.BlockSpec(memory_space=pltpu.SMEM),
            pl.BlockSpec(memory_space=pltpu.SMEM),
        ],
        out_specs=pl.BlockSpec(memory_space=pltpu.VMEM),
        scratch_shapes=[
            pltpu.VMEM((N_HOPS, SUBS, rows, k), x.dtype),
            pltpu.VMEM((N_HOPS, SUBS, rows, k), x.dtype),
            pltpu.SemaphoreType.DMA((N_HOPS, SUBS)),
            pltpu.SemaphoreType.DMA((N_HOPS, SUBS)),
            pltpu.SemaphoreType.DMA((N_HOPS, SUBS)),
            pltpu.SemaphoreType.DMA((N_HOPS, SUBS)),
        ],
        compiler_params=pltpu.CompilerParams(collective_id=0),
    )(x, w_mat, scale_x, scale_w, nbrs, cw_org, ccw_org)


# device time: 102176 ns/iter; 2.2008x vs baseline; 1.0551x over previous
import jax
import jax.numpy as jnp
from jax import lax
from jax.experimental import pallas as pl
from jax.experimental.pallas import tpu as pltpu

N_DEV = 16
N_HOPS = 8
SUBS = 4


def _perm_of(t):
    c = t // 4
    u = t % 4
    z = jnp.where(c % 2 == 0, u, 3 - u)
    return z * 4 + c


def kernel(x, w_mat, scale_x, scale_w):
    m_per, k = x.shape
    _, n_per = w_mat.shape
    m_total = N_DEV * m_per
    rows = m_per // SUBS

    x = x.astype(jnp.float8_e4m3fn)
    w_mat = w_mat.astype(jnp.float8_e4m3fn)

    my = lax.axis_index("i")
    mz = my // 4
    mc = my % 4
    r = 4 * mc + jnp.where(mc % 2 == 0, mz, 3 - mz)
    nbrs = jnp.stack([_perm_of((r + N_DEV - 1) % N_DEV),
                      _perm_of((r + 1) % N_DEV)])
    hs = jnp.arange(N_HOPS, dtype=jnp.int32)
    cw_org = _perm_of((r + N_DEV - 1 - hs) % N_DEV)
    ccw_org = _perm_of((r + 1 + hs) % N_DEV)

    def cw_has(h, s):
        return h < N_HOPS - 1 or s < SUBS // 2

    def ccw_has(h, s):
        return h < N_HOPS - 1 or s >= SUBS // 2

    def body(x_ref, w_ref, sx_ref, sw_ref, nbr_ref, cw_org_ref, ccw_org_ref,
             out_ref, cw_ref, ccw_ref, cw_send, cw_recv, ccw_send, ccw_recv):
        left = nbr_ref[0]
        right = nbr_ref[1]

        barrier_sem = pltpu.get_barrier_semaphore()
        for nbr in (left, right):
            pl.semaphore_signal(
                barrier_sem, inc=1,
                device_id=(nbr,), device_id_type=pl.DeviceIdType.MESH,
            )
        pl.semaphore_wait(barrier_sem, 2)

        scale = sx_ref[0] * sw_ref[0]

        def gemm_store(chunk, row_start):
            acc = lax.dot_general(
                chunk, w_ref[...],
                (((1,), (0,)), ((), ())),
                preferred_element_type=jnp.float32,
            )
            y = acc * scale
            z = y / (1.0 + jnp.exp(-jnp.clip(y, -60.0, 60.0)))
            out_ref[pl.ds(row_start, chunk.shape[0]), :] = z

        def mk(src, dst_buf, h, s, send_sems, recv_sems, dev):
            return pltpu.make_async_remote_copy(
                src_ref=src,
                dst_ref=dst_buf.at[h, s],
                send_sem=send_sems.at[h, s],
                recv_sem=recv_sems.at[h, s],
                device_id=(dev,),
                device_id_type=pl.DeviceIdType.MESH,
            )

        sends = []

        for s in range(SUBS):
            piece = x_ref.at[pl.ds(s * rows, rows)]
            rd = mk(piece, cw_ref, 0, s, cw_send, cw_recv, right)
            rd.start()
            sends.append(rd)
            rd = mk(piece, ccw_ref, 0, s, ccw_send, ccw_recv, left)
            rd.start()
            sends.append(rd)
        gemm_store(x_ref[...], lax.axis_index("i") * m_per)

        def recv_cw(h, s):
            return mk(x_ref.at[pl.ds(0, rows)], cw_ref, h, s,
                      cw_send, cw_recv, right)

        def recv_ccw(h, s):
            return mk(x_ref.at[pl.ds(0, rows)], ccw_ref, h, s,
                      ccw_send, ccw_recv, left)

        for h in range(N_HOPS):
            for s in range(SUBS):
                if cw_has(h, s):
                    recv_cw(h, s).wait_recv()
                    if h + 1 < N_HOPS and cw_has(h + 1, s):
                        rd = mk(cw_ref.at[h, s], cw_ref, h + 1, s,
                                cw_send, cw_recv, right)
                        rd.start()
                        sends.append(rd)
                if ccw_has(h, s):
                    recv_ccw(h, s).wait_recv()
                    if h + 1 < N_HOPS and ccw_has(h + 1, s):
                        rd = mk(ccw_ref.at[h, s], ccw_ref, h + 1, s,
                                ccw_send, ccw_recv, left)
                        rd.start()
                        sends.append(rd)
            for s in range(SUBS):
                if cw_has(h, s):
                    gemm_store(cw_ref[h, s], cw_org_ref[h] * m_per + s * rows)
                if ccw_has(h, s):
                    gemm_store(ccw_ref[h, s], ccw_org_ref[h] * m_per + s * rows)

        for rd in sends:
            rd.wait_send()

    return pl.pallas_call(
        body,
        out_shape=jax.ShapeDtypeStruct((m_total, n_per), jnp.float32),
        in_specs=[
            pl.BlockSpec(memory_space=pltpu.VMEM),
            pl.BlockSpec(memory_space=pltpu.VMEM),
            pl.BlockSpec(memory_space=pltpu.SMEM),
            pl.BlockSpec(memory_space=pltpu.SMEM),
            pl.BlockSpec(memory_space=pltpu.SMEM),
            pl.BlockSpec(memory_space=pltpu.SMEM),
            pl.BlockSpec(memory_space=pltpu.SMEM),
        ],
        out_specs=pl.BlockSpec(memory_space=pltpu.VMEM),
        scratch_shapes=[
            pltpu.VMEM((N_HOPS, SUBS, rows, k), x.dtype),
            pltpu.VMEM((N_HOPS, SUBS, rows, k), x.dtype),
            pltpu.SemaphoreType.DMA((N_HOPS, SUBS)),
            pltpu.SemaphoreType.DMA((N_HOPS, SUBS)),
            pltpu.SemaphoreType.DMA((N_HOPS, SUBS)),
            pltpu.SemaphoreType.DMA((N_HOPS, SUBS)),
        ],
        compiler_params=pltpu.CompilerParams(collective_id=0),
    )(x, w_mat, scale_x, scale_w, nbrs, cw_org, ccw_org)
